# baseline (speedup 1.0000x reference)
import functools

import jax
import jax.numpy as jnp
from jax import lax
from jax.experimental import pallas as pl
from jax.experimental.pallas import tpu as pltpu
from jax.experimental.pallas import tpu_sc as plsc

L = 16
NBINS = 256
ROWS = 32


def _build(n_planes, H, W):
    assert n_planes == 48 and H == 512
    n_chunks_full = H // ROWS
    n_chunks_half = n_chunks_full // 2
    total_px = H * W
    vregs_per_row = W // L
    bank_words = L * NBINS

    mesh = plsc.VectorSubcoreMesh(core_axis_name="c", subcore_axis_name="s")

    @functools.partial(
        pl.kernel,
        out_type=jax.ShapeDtypeStruct((n_planes, H, W), jnp.float32),
        mesh=mesh,
        compiler_params=pltpu.CompilerParams(needs_layout_passes=False),
        scratch_types=[
            pltpu.VMEM((ROWS, W), jnp.float32),
            pltpu.VMEM((ROWS, W), jnp.float32),
            pltpu.VMEM((ROWS, W), jnp.float32),
            pltpu.VMEM((ROWS, W), jnp.float32),
            pltpu.VMEM((bank_words,), jnp.int32),
            pltpu.VMEM((NBINS * L,), jnp.float32),
            pltpu.SemaphoreType.DMA,
            pltpu.SemaphoreType.DMA,
            pltpu.SemaphoreType.DMA,
            pltpu.SemaphoreType.DMA,
        ],
    )
    def eq_kernel(x_hbm, o_hbm, in0, in1, out0, out1, hist_v, lut_v,
                  si0, si1, so0, so1):
        c = lax.axis_index("c")
        s = lax.axis_index("s")

        in_bufs = (in0, in1)
        in_sems = (si0, si1)
        out_bufs = (out0, out1)
        out_sems = (so0, so1)

        iota = lax.broadcasted_iota(jnp.int32, (L,), 0)
        ones = jnp.full((L,), 1, jnp.int32)
        zero = jnp.full((L,), 0, jnp.int32)

        def zero_banks():
            def zbody(i, _):
                hist_v[pl.ds(i * L, L)] = zero
                return 0
            lax.fori_loop(0, bank_words // L, zbody, 0)

        def hist_pass(plane, row_base, n_slots):
            def src_of(t):
                return x_hbm.at[plane, pl.ds(row_base + t * ROWS, ROWS)]

            pltpu.async_copy(src_of(0), in_bufs[0], in_sems[0])
            pltpu.async_copy(src_of(1), in_bufs[1], in_sems[1])

            def steps(t2, _):
                for b in range(2):
                    t = t2 * 2 + b
                    buf, sem = in_bufs[b], in_sems[b]
                    pltpu.make_async_copy(src_of(t), buf, sem).wait()

                    def hrow(r, _):
                        for u in range(vregs_per_row):
                            v = buf[r, pl.ds(u * L, L)]
                            vi = jnp.clip(v, 0.0, 255.0).astype(jnp.int32)
                            plsc.addupdate_scatter(
                                hist_v, [vi * L + iota], ones)
                        return 0
                    lax.fori_loop(0, ROWS, hrow, 0)

                    @pl.when(t + 2 < n_slots)
                    def _():
                        pltpu.async_copy(src_of(t + 2), buf, sem)
                return 0
            lax.fori_loop(0, n_slots // 2, steps, 0)

        def build_lut():
            def scanbins(b, hlast):
                sb = jnp.sum(hist_v[pl.ds(b * L, L)])
                return jnp.where(sb > 0, sb, hlast)
            hlast = lax.fori_loop(0, NBINS, scanbins, jnp.int32(0))

            step = (jnp.int32(total_px) - hlast) // 255
            den = jnp.maximum(step, 1)
            half_step = step // 2
            is0 = step == 0

            def lutb(b, run):
                sb = jnp.sum(hist_v[pl.ds(b * L, L)])
                lv = jnp.clip((run + half_step) // den, 0, 255)
                lv = jnp.where(is0, b, lv)
                lut_v[pl.ds(b * L, L)] = lax.broadcast(
                    lv.astype(jnp.float32), (L,))
                return run + sb
            lax.fori_loop(0, NBINS, lutb, jnp.int32(0))

        def apply_pass(plane, row_base, n_slots):
            def src_of(t):
                return x_hbm.at[plane, pl.ds(row_base + t * ROWS, ROWS)]

            def dst_of(t):
                return o_hbm.at[plane, pl.ds(row_base + t * ROWS, ROWS)]

            pltpu.async_copy(src_of(0), in_bufs[0], in_sems[0])
            pltpu.async_copy(src_of(1), in_bufs[1], in_sems[1])

            def steps(t2, _):
                for b in range(2):
                    t = t2 * 2 + b
                    buf, sem = in_bufs[b], in_sems[b]
                    obuf, osem = out_bufs[b], out_sems[b]
                    pltpu.make_async_copy(src_of(t), buf, sem).wait()

                    @pl.when(t >= 2)
                    def _():
                        pltpu.make_async_copy(obuf, dst_of(t), osem).wait()

                    def arow(r, _):
                        for u in range(vregs_per_row):
                            v = buf[r, pl.ds(u * L, L)]
                            vi = jnp.clip(v, 0.0, 255.0).astype(jnp.int32)
                            obuf[r, pl.ds(u * L, L)] = plsc.load_gather(
                                lut_v, [vi * L + iota])
                        return 0
                    lax.fori_loop(0, ROWS, arow, 0)

                    pltpu.async_copy(obuf, dst_of(t), osem)

                    @pl.when(t + 2 < n_slots)
                    def _():
                        pltpu.async_copy(src_of(t + 2), buf, sem)
                return 0
            lax.fori_loop(0, n_slots // 2, steps, 0)

            for b in range(2):
                pltpu.make_async_copy(
                    out_bufs[b], dst_of(n_slots - 2 + b), out_sems[b]).wait()

        plane_a = c * 24 + s
        zero_banks()
        hist_pass(plane_a, 0, n_chunks_full)
        build_lut()
        apply_pass(plane_a, 0, n_chunks_full)

        plane_b = c * 24 + 16 + s // 2
        half_base = (s % 2) * (H // 2)
        zero_banks()
        hist_pass(plane_b, 0, n_chunks_full)
        build_lut()
        apply_pass(plane_b, half_base, n_chunks_half)

    return eq_kernel


def kernel(x, magnitude):
    B, H, W, C = x.shape
    xp = jnp.transpose(x, (0, 3, 1, 2)).reshape(B * C, H, W)
    eq = _build(B * C, H, W)
    out = eq(xp.astype(jnp.float32))
    return out.reshape(B, C, H, W).transpose(0, 2, 3, 1).astype(x.dtype)


# speedup vs baseline: 2.8838x; 2.8838x over previous
import functools

import jax
import jax.numpy as jnp
from jax import lax
from jax.experimental import pallas as pl
from jax.experimental.pallas import tpu as pltpu
from jax.experimental.pallas import tpu_sc as plsc

L = 16
NBINS = 256
ROWS = 32


def _build(n_planes, H, W):
    assert n_planes == 48 and H == 512
    n_chunks_full = H // ROWS
    n_chunks_half = n_chunks_full // 2
    total_px = H * W
    vregs_per_row = W // L
    bank_words = L * NBINS

    mesh = plsc.VectorSubcoreMesh(core_axis_name="c", subcore_axis_name="s")

    @functools.partial(
        pl.kernel,
        out_type=jax.ShapeDtypeStruct((n_planes, H, W), jnp.float32),
        mesh=mesh,
        compiler_params=pltpu.CompilerParams(needs_layout_passes=False),
        scratch_types=[
            pltpu.VMEM((ROWS, W), jnp.float32),
            pltpu.VMEM((ROWS, W), jnp.float32),
            pltpu.VMEM((ROWS, W), jnp.float32),
            pltpu.VMEM((ROWS, W), jnp.float32),
            pltpu.VMEM((bank_words,), jnp.int32),
            pltpu.VMEM((NBINS * L,), jnp.float32),
            pltpu.SemaphoreType.DMA,
            pltpu.SemaphoreType.DMA,
            pltpu.SemaphoreType.DMA,
            pltpu.SemaphoreType.DMA,
        ],
    )
    def eq_kernel(x_hbm, o_hbm, in0, in1, out0, out1, hist_v, lut_v,
                  si0, si1, so0, so1):
        c = lax.axis_index("c")
        s = lax.axis_index("s")

        in_bufs = (in0, in1)
        in_sems = (si0, si1)
        out_bufs = (out0, out1)
        out_sems = (so0, so1)

        iota = lax.broadcasted_iota(jnp.int32, (L,), 0)
        ones = jnp.full((L,), 1, jnp.int32)
        zero = jnp.full((L,), 0, jnp.int32)

        def zero_banks():
            def zbody(i, _):
                hist_v[pl.ds(i * L, L)] = zero
                return 0
            lax.fori_loop(0, bank_words // L, zbody, 0)

        def hist_pass(plane, row_base, n_slots):
            def src_of(t):
                return x_hbm.at[plane, pl.ds(row_base + t * ROWS, ROWS)]

            pltpu.async_copy(src_of(0), in_bufs[0], in_sems[0])
            pltpu.async_copy(src_of(1), in_bufs[1], in_sems[1])

            def steps(t2, _):
                for b in range(2):
                    t = t2 * 2 + b
                    buf, sem = in_bufs[b], in_sems[b]
                    pltpu.make_async_copy(src_of(t), buf, sem).wait()

                    @plsc.parallel_loop(0, ROWS, unroll=1)
                    def hrow(r):
                        for u in range(vregs_per_row):
                            v = buf[r, pl.ds(u * L, L)]
                            vi = jnp.clip(v, 0.0, 255.0).astype(jnp.int32)
                            plsc.addupdate_scatter(
                                hist_v, [vi * L + iota], ones)

                    @pl.when(t + 2 < n_slots)
                    def _():
                        pltpu.async_copy(src_of(t + 2), buf, sem)
                return 0
            lax.fori_loop(0, n_slots // 2, steps, 0)

        def build_lut():
            def scanbins(b, hlast):
                sb = jnp.sum(hist_v[pl.ds(b * L, L)])
                return jnp.where(sb > 0, sb, hlast)
            hlast = lax.fori_loop(0, NBINS, scanbins, jnp.int32(0))

            step = (jnp.int32(total_px) - hlast) // 255
            den = jnp.maximum(step, 1)
            half_step = step // 2
            is0 = step == 0

            def lutb(b, run):
                sb = jnp.sum(hist_v[pl.ds(b * L, L)])
                lv = jnp.clip((run + half_step) // den, 0, 255)
                lv = jnp.where(is0, b, lv)
                lut_v[pl.ds(b * L, L)] = lax.broadcast(
                    lv.astype(jnp.float32), (L,))
                return run + sb
            lax.fori_loop(0, NBINS, lutb, jnp.int32(0))

        def apply_pass(plane, row_base, n_slots):
            def src_of(t):
                return x_hbm.at[plane, pl.ds(row_base + t * ROWS, ROWS)]

            def dst_of(t):
                return o_hbm.at[plane, pl.ds(row_base + t * ROWS, ROWS)]

            pltpu.async_copy(src_of(0), in_bufs[0], in_sems[0])
            pltpu.async_copy(src_of(1), in_bufs[1], in_sems[1])

            def steps(t2, _):
                for b in range(2):
                    t = t2 * 2 + b
                    buf, sem = in_bufs[b], in_sems[b]
                    obuf, osem = out_bufs[b], out_sems[b]
                    pltpu.make_async_copy(src_of(t), buf, sem).wait()

                    @pl.when(t >= 2)
                    def _():
                        pltpu.make_async_copy(obuf, dst_of(t), osem).wait()

                    @plsc.parallel_loop(0, ROWS, unroll=1)
                    def arow(r):
                        for u in range(vregs_per_row):
                            v = buf[r, pl.ds(u * L, L)]
                            vi = jnp.clip(v, 0.0, 255.0).astype(jnp.int32)
                            obuf[r, pl.ds(u * L, L)] = plsc.load_gather(
                                lut_v, [vi * L + iota])

                    pltpu.async_copy(obuf, dst_of(t), osem)

                    @pl.when(t + 2 < n_slots)
                    def _():
                        pltpu.async_copy(src_of(t + 2), buf, sem)
                return 0
            lax.fori_loop(0, n_slots // 2, steps, 0)

            for b in range(2):
                pltpu.make_async_copy(
                    out_bufs[b], dst_of(n_slots - 2 + b), out_sems[b]).wait()

        plane_a = c * 24 + s
        zero_banks()
        hist_pass(plane_a, 0, n_chunks_full)
        build_lut()
        apply_pass(plane_a, 0, n_chunks_full)

        plane_b = c * 24 + 16 + s // 2
        half_base = (s % 2) * (H // 2)
        zero_banks()
        hist_pass(plane_b, 0, n_chunks_full)
        build_lut()
        apply_pass(plane_b, half_base, n_chunks_half)

    return eq_kernel


def kernel(x, magnitude):
    B, H, W, C = x.shape
    xp = jnp.transpose(x, (0, 3, 1, 2)).reshape(B * C, H, W)
    eq = _build(B * C, H, W)
    out = eq(xp.astype(jnp.float32))
    return out.reshape(B, C, H, W).transpose(0, 2, 3, 1).astype(x.dtype)


# speedup vs baseline: 3.3839x; 1.1734x over previous
import functools

import jax
import jax.numpy as jnp
from jax import lax
from jax.experimental import pallas as pl
from jax.experimental.pallas import tpu as pltpu
from jax.experimental.pallas import tpu_sc as plsc

L = 16
NBINS = 256
ROWS = 32


def _build(n_planes, H, W):
    assert n_planes == 48 and H == 512
    n_chunks_full = H // ROWS
    n_chunks_half = n_chunks_full // 2
    total_px = H * W
    vregs_per_row = W // L
    bank_words = L * NBINS

    mesh = plsc.VectorSubcoreMesh(core_axis_name="c", subcore_axis_name="s")

    @functools.partial(
        pl.kernel,
        out_type=jax.ShapeDtypeStruct((n_planes, H, W), jnp.float32),
        mesh=mesh,
        compiler_params=pltpu.CompilerParams(needs_layout_passes=False),
        scratch_types=[
            pltpu.VMEM((ROWS, W), jnp.float32),
            pltpu.VMEM((ROWS, W), jnp.float32),
            pltpu.VMEM((ROWS, W), jnp.float32),
            pltpu.VMEM((ROWS, W), jnp.float32),
            pltpu.VMEM((bank_words,), jnp.int32),
            pltpu.VMEM((NBINS * L,), jnp.float32),
            pltpu.SemaphoreType.DMA,
            pltpu.SemaphoreType.DMA,
            pltpu.SemaphoreType.DMA,
            pltpu.SemaphoreType.DMA,
        ],
    )
    def eq_kernel(x_hbm, o_hbm, in0, in1, out0, out1, hist_v, lut_v,
                  si0, si1, so0, so1):
        c = lax.axis_index("c")
        s = lax.axis_index("s")

        in_bufs = (in0, in1)
        in_sems = (si0, si1)
        out_bufs = (out0, out1)
        out_sems = (so0, so1)

        iota = lax.broadcasted_iota(jnp.int32, (L,), 0)
        ones = jnp.full((L,), 1, jnp.int32)
        zero = jnp.full((L,), 0, jnp.int32)

        def zero_banks():
            def zbody(i, _):
                hist_v[pl.ds(i * L, L)] = zero
                return 0
            lax.fori_loop(0, bank_words // L, zbody, 0)

        def hist_pass(plane, row_base, n_slots):
            def src_of(t):
                return x_hbm.at[plane, pl.ds(row_base + t * ROWS, ROWS)]

            pltpu.async_copy(src_of(0), in_bufs[0], in_sems[0])
            pltpu.async_copy(src_of(1), in_bufs[1], in_sems[1])

            def steps(t2, _):
                for b in range(2):
                    t = t2 * 2 + b
                    buf, sem = in_bufs[b], in_sems[b]
                    pltpu.make_async_copy(src_of(t), buf, sem).wait()

                    @plsc.parallel_loop(0, ROWS * 2, unroll=2)
                    def hrow(h):
                        r = h >> 1
                        u0 = (h & 1) * (vregs_per_row // 2)
                        for u in range(vregs_per_row // 2):
                            v = buf[r, pl.ds((u0 + u) * L, L)]
                            vi = jnp.clip(v, 0.0, 255.0).astype(jnp.int32)
                            plsc.addupdate_scatter(
                                hist_v, [vi * L + iota], ones)

                    @pl.when(t + 2 < n_slots)
                    def _():
                        pltpu.async_copy(src_of(t + 2), buf, sem)
                return 0
            lax.fori_loop(0, n_slots // 2, steps, 0)

        def build_lut():
            def scanbins(b, hlast):
                sb = jnp.sum(hist_v[pl.ds(b * L, L)])
                return jnp.where(sb > 0, sb, hlast)
            hlast = lax.fori_loop(0, NBINS, scanbins, jnp.int32(0))

            step = (jnp.int32(total_px) - hlast) // 255
            den = jnp.maximum(step, 1)
            half_step = step // 2
            is0 = step == 0

            def lutb(b, run):
                sb = jnp.sum(hist_v[pl.ds(b * L, L)])
                lv = jnp.clip((run + half_step) // den, 0, 255)
                lv = jnp.where(is0, b, lv)
                lut_v[pl.ds(b * L, L)] = lax.broadcast(
                    lv.astype(jnp.float32), (L,))
                return run + sb
            lax.fori_loop(0, NBINS, lutb, jnp.int32(0))

        def apply_pass(plane, row_base, n_slots):
            def src_of(t):
                return x_hbm.at[plane, pl.ds(row_base + t * ROWS, ROWS)]

            def dst_of(t):
                return o_hbm.at[plane, pl.ds(row_base + t * ROWS, ROWS)]

            pltpu.async_copy(src_of(0), in_bufs[0], in_sems[0])
            pltpu.async_copy(src_of(1), in_bufs[1], in_sems[1])

            def steps(t2, _):
                for b in range(2):
                    t = t2 * 2 + b
                    buf, sem = in_bufs[b], in_sems[b]
                    obuf, osem = out_bufs[b], out_sems[b]
                    pltpu.make_async_copy(src_of(t), buf, sem).wait()

                    @pl.when(t >= 2)
                    def _():
                        pltpu.make_async_copy(obuf, dst_of(t), osem).wait()

                    @plsc.parallel_loop(0, ROWS * 2, unroll=2)
                    def arow(h):
                        r = h >> 1
                        u0 = (h & 1) * (vregs_per_row // 2)
                        for u in range(vregs_per_row // 2):
                            v = buf[r, pl.ds((u0 + u) * L, L)]
                            vi = jnp.clip(v, 0.0, 255.0).astype(jnp.int32)
                            obuf[r, pl.ds((u0 + u) * L, L)] = plsc.load_gather(
                                lut_v, [vi * L + iota])

                    pltpu.async_copy(obuf, dst_of(t), osem)

                    @pl.when(t + 2 < n_slots)
                    def _():
                        pltpu.async_copy(src_of(t + 2), buf, sem)
                return 0
            lax.fori_loop(0, n_slots // 2, steps, 0)

            for b in range(2):
                pltpu.make_async_copy(
                    out_bufs[b], dst_of(n_slots - 2 + b), out_sems[b]).wait()

        plane_a = c * 24 + s
        zero_banks()
        hist_pass(plane_a, 0, n_chunks_full)
        build_lut()
        apply_pass(plane_a, 0, n_chunks_full)

        plane_b = c * 24 + 16 + s // 2
        half_base = (s % 2) * (H // 2)
        zero_banks()
        hist_pass(plane_b, 0, n_chunks_full)
        build_lut()
        apply_pass(plane_b, half_base, n_chunks_half)

    return eq_kernel


def kernel(x, magnitude):
    B, H, W, C = x.shape
    xp = jnp.transpose(x, (0, 3, 1, 2)).reshape(B * C, H, W)
    eq = _build(B * C, H, W)
    out = eq(xp.astype(jnp.float32))
    return out.reshape(B, C, H, W).transpose(0, 2, 3, 1).astype(x.dtype)


# speedup vs baseline: 3.6308x; 1.0730x over previous
import functools

import jax
import jax.numpy as jnp
from jax import lax
from jax.experimental import pallas as pl
from jax.experimental.pallas import tpu as pltpu
from jax.experimental.pallas import tpu_sc as plsc

L = 16
NBINS = 256
ROWS = 32


def _build(n_planes, H, W):
    assert n_planes == 48 and H == 512
    n_chunks_full = H // ROWS
    n_chunks_half = n_chunks_full // 2
    total_px = H * W
    vregs_per_row = W // L
    bank_words = L * NBINS

    mesh = plsc.VectorSubcoreMesh(core_axis_name="c", subcore_axis_name="s")

    @functools.partial(
        pl.kernel,
        out_type=jax.ShapeDtypeStruct((n_planes, H, W), jnp.float32),
        mesh=mesh,
        compiler_params=pltpu.CompilerParams(needs_layout_passes=False),
        scratch_types=[
            pltpu.VMEM((ROWS, W), jnp.float32),
            pltpu.VMEM((ROWS, W), jnp.float32),
            pltpu.VMEM((ROWS, W), jnp.float32),
            pltpu.VMEM((ROWS, W), jnp.float32),
            pltpu.VMEM((bank_words,), jnp.int32),
            pltpu.VMEM((NBINS * L,), jnp.float32),
            pltpu.SemaphoreType.DMA,
            pltpu.SemaphoreType.DMA,
            pltpu.SemaphoreType.DMA,
            pltpu.SemaphoreType.DMA,
        ],
    )
    def eq_kernel(x_hbm, o_hbm, in0, in1, out0, out1, hist_v, lut_v,
                  si0, si1, so0, so1):
        c = lax.axis_index("c")
        s = lax.axis_index("s")

        in_bufs = (in0, in1)
        in_sems = (si0, si1)
        out_bufs = (out0, out1)
        out_sems = (so0, so1)

        iota = lax.broadcasted_iota(jnp.int32, (L,), 0)
        ones = jnp.full((L,), 1, jnp.int32)
        zero = jnp.full((L,), 0, jnp.int32)

        def zero_banks():
            def zbody(i, _):
                hist_v[pl.ds(i * L, L)] = zero
                return 0
            lax.fori_loop(0, bank_words // L, zbody, 0)

        def hist_pass(plane, row_base, n_slots):
            def src_of(t):
                return x_hbm.at[plane, pl.ds(row_base + t * ROWS, ROWS)]

            pltpu.async_copy(src_of(0), in_bufs[0], in_sems[0])
            pltpu.async_copy(src_of(1), in_bufs[1], in_sems[1])

            def steps(t2, _):
                for b in range(2):
                    t = t2 * 2 + b
                    buf, sem = in_bufs[b], in_sems[b]
                    pltpu.make_async_copy(src_of(t), buf, sem).wait()

                    @plsc.parallel_loop(0, ROWS * 4, unroll=4)
                    def hrow(h):
                        r = h >> 2
                        u0 = (h & 3) * (vregs_per_row // 4)
                        for u in range(vregs_per_row // 4):
                            v = buf[r, pl.ds((u0 + u) * L, L)]
                            vi = jnp.clip(v, 0.0, 255.0).astype(jnp.int32)
                            plsc.addupdate_scatter(
                                hist_v, [vi * L + iota], ones)

                    @pl.when(t + 2 < n_slots)
                    def _():
                        pltpu.async_copy(src_of(t + 2), buf, sem)
                return 0
            lax.fori_loop(0, n_slots // 2, steps, 0)

        def build_lut():
            def scanbins(b, hlast):
                sb = jnp.sum(hist_v[pl.ds(b * L, L)])
                return jnp.where(sb > 0, sb, hlast)
            hlast = lax.fori_loop(0, NBINS, scanbins, jnp.int32(0))

            step = (jnp.int32(total_px) - hlast) // 255
            den = jnp.maximum(step, 1)
            half_step = step // 2
            is0 = step == 0

            def lutb(b, run):
                sb = jnp.sum(hist_v[pl.ds(b * L, L)])
                lv = jnp.clip((run + half_step) // den, 0, 255)
                lv = jnp.where(is0, b, lv)
                lut_v[pl.ds(b * L, L)] = lax.broadcast(
                    lv.astype(jnp.float32), (L,))
                return run + sb
            lax.fori_loop(0, NBINS, lutb, jnp.int32(0))

        def apply_pass(plane, row_base, n_slots):
            def src_of(t):
                return x_hbm.at[plane, pl.ds(row_base + t * ROWS, ROWS)]

            def dst_of(t):
                return o_hbm.at[plane, pl.ds(row_base + t * ROWS, ROWS)]

            pltpu.async_copy(src_of(0), in_bufs[0], in_sems[0])
            pltpu.async_copy(src_of(1), in_bufs[1], in_sems[1])

            def steps(t2, _):
                for b in range(2):
                    t = t2 * 2 + b
                    buf, sem = in_bufs[b], in_sems[b]
                    obuf, osem = out_bufs[b], out_sems[b]
                    pltpu.make_async_copy(src_of(t), buf, sem).wait()

                    @pl.when(t >= 2)
                    def _():
                        pltpu.make_async_copy(obuf, dst_of(t), osem).wait()

                    @plsc.parallel_loop(0, ROWS * 4, unroll=4)
                    def arow(h):
                        r = h >> 2
                        u0 = (h & 3) * (vregs_per_row // 4)
                        for u in range(vregs_per_row // 4):
                            v = buf[r, pl.ds((u0 + u) * L, L)]
                            vi = jnp.clip(v, 0.0, 255.0).astype(jnp.int32)
                            obuf[r, pl.ds((u0 + u) * L, L)] = plsc.load_gather(
                                lut_v, [vi * L + iota])

                    pltpu.async_copy(obuf, dst_of(t), osem)

                    @pl.when(t + 2 < n_slots)
                    def _():
                        pltpu.async_copy(src_of(t + 2), buf, sem)
                return 0
            lax.fori_loop(0, n_slots // 2, steps, 0)

            for b in range(2):
                pltpu.make_async_copy(
                    out_bufs[b], dst_of(n_slots - 2 + b), out_sems[b]).wait()

        plane_a = c * 24 + s
        zero_banks()
        hist_pass(plane_a, 0, n_chunks_full)
        build_lut()
        apply_pass(plane_a, 0, n_chunks_full)

        plane_b = c * 24 + 16 + s // 2
        half_base = (s % 2) * (H // 2)
        zero_banks()
        hist_pass(plane_b, 0, n_chunks_full)
        build_lut()
        apply_pass(plane_b, half_base, n_chunks_half)

    return eq_kernel


def kernel(x, magnitude):
    B, H, W, C = x.shape
    xp = jnp.transpose(x, (0, 3, 1, 2)).reshape(B * C, H, W)
    eq = _build(B * C, H, W)
    out = eq(xp.astype(jnp.float32))
    return out.reshape(B, C, H, W).transpose(0, 2, 3, 1).astype(x.dtype)


# speedup vs baseline: 3.8719x; 1.0664x over previous
import functools

import jax
import jax.numpy as jnp
from jax import lax
from jax.experimental import pallas as pl
from jax.experimental.pallas import tpu as pltpu
from jax.experimental.pallas import tpu_sc as plsc

L = 16
NBINS = 256
ROWS = 32


def _build(n_planes, H, W):
    assert n_planes == 48 and H == 512
    n_chunks_full = H // ROWS
    n_chunks_half = n_chunks_full // 2
    total_px = H * W
    vregs_per_row = W // L
    bank_words = L * NBINS

    mesh = plsc.VectorSubcoreMesh(core_axis_name="c", subcore_axis_name="s")

    @functools.partial(
        pl.kernel,
        out_type=jax.ShapeDtypeStruct((n_planes, H, W), jnp.float32),
        mesh=mesh,
        compiler_params=pltpu.CompilerParams(needs_layout_passes=False),
        scratch_types=[
            pltpu.VMEM((ROWS, W), jnp.float32),
            pltpu.VMEM((ROWS, W), jnp.float32),
            pltpu.VMEM((ROWS, W), jnp.float32),
            pltpu.VMEM((ROWS, W), jnp.float32),
            pltpu.VMEM((bank_words,), jnp.int32),
            pltpu.VMEM((NBINS * L,), jnp.float32),
            pltpu.SemaphoreType.DMA,
            pltpu.SemaphoreType.DMA,
            pltpu.SemaphoreType.DMA,
            pltpu.SemaphoreType.DMA,
        ],
    )
    def eq_kernel(x_hbm, o_hbm, in0, in1, out0, out1, hist_v, lut_v,
                  si0, si1, so0, so1):
        c = lax.axis_index("c")
        s = lax.axis_index("s")

        in_bufs = (in0, in1)
        in_sems = (si0, si1)
        out_bufs = (out0, out1)
        out_sems = (so0, so1)

        iota = lax.broadcasted_iota(jnp.int32, (L,), 0)
        ones = jnp.full((L,), 1, jnp.int32)
        zero = jnp.full((L,), 0, jnp.int32)

        def zero_banks():
            def zbody(i, _):
                hist_v[pl.ds(i * L, L)] = zero
                return 0
            lax.fori_loop(0, bank_words // L, zbody, 0)

        def hist_pass(plane, row_base, n_slots):
            def src_of(t):
                return x_hbm.at[plane, pl.ds(row_base + t * ROWS, ROWS)]

            pltpu.async_copy(src_of(0), in_bufs[0], in_sems[0])
            pltpu.async_copy(src_of(1), in_bufs[1], in_sems[1])

            def steps(t2, _):
                for b in range(2):
                    t = t2 * 2 + b
                    buf, sem = in_bufs[b], in_sems[b]
                    pltpu.make_async_copy(src_of(t), buf, sem).wait()

                    @plsc.parallel_loop(0, ROWS * 8, unroll=8)
                    def hrow(h):
                        r = h >> 3
                        u0 = (h & 7) * (vregs_per_row // 8)
                        for u in range(vregs_per_row // 8):
                            v = buf[r, pl.ds((u0 + u) * L, L)]
                            vi = jnp.clip(v, 0.0, 255.0).astype(jnp.int32)
                            plsc.addupdate_scatter(
                                hist_v, [vi * L + iota], ones)

                    @pl.when(t + 2 < n_slots)
                    def _():
                        pltpu.async_copy(src_of(t + 2), buf, sem)
                return 0
            lax.fori_loop(0, n_slots // 2, steps, 0)

        def build_lut():
            def scanbins(b, hlast):
                sb = jnp.sum(hist_v[pl.ds(b * L, L)])
                return jnp.where(sb > 0, sb, hlast)
            hlast = lax.fori_loop(0, NBINS, scanbins, jnp.int32(0))

            step = (jnp.int32(total_px) - hlast) // 255
            den = jnp.maximum(step, 1)
            half_step = step // 2
            is0 = step == 0

            def lutb(b, run):
                sb = jnp.sum(hist_v[pl.ds(b * L, L)])
                lv = jnp.clip((run + half_step) // den, 0, 255)
                lv = jnp.where(is0, b, lv)
                lut_v[pl.ds(b * L, L)] = lax.broadcast(
                    lv.astype(jnp.float32), (L,))
                return run + sb
            lax.fori_loop(0, NBINS, lutb, jnp.int32(0))

        def apply_pass(plane, row_base, n_slots):
            def src_of(t):
                return x_hbm.at[plane, pl.ds(row_base + t * ROWS, ROWS)]

            def dst_of(t):
                return o_hbm.at[plane, pl.ds(row_base + t * ROWS, ROWS)]

            pltpu.async_copy(src_of(0), in_bufs[0], in_sems[0])
            pltpu.async_copy(src_of(1), in_bufs[1], in_sems[1])

            def steps(t2, _):
                for b in range(2):
                    t = t2 * 2 + b
                    buf, sem = in_bufs[b], in_sems[b]
                    obuf, osem = out_bufs[b], out_sems[b]
                    pltpu.make_async_copy(src_of(t), buf, sem).wait()

                    @pl.when(t >= 2)
                    def _():
                        pltpu.make_async_copy(obuf, dst_of(t), osem).wait()

                    @plsc.parallel_loop(0, ROWS * 8, unroll=8)
                    def arow(h):
                        r = h >> 3
                        u0 = (h & 7) * (vregs_per_row // 8)
                        for u in range(vregs_per_row // 8):
                            v = buf[r, pl.ds((u0 + u) * L, L)]
                            vi = jnp.clip(v, 0.0, 255.0).astype(jnp.int32)
                            obuf[r, pl.ds((u0 + u) * L, L)] = plsc.load_gather(
                                lut_v, [vi * L + iota])

                    pltpu.async_copy(obuf, dst_of(t), osem)

                    @pl.when(t + 2 < n_slots)
                    def _():
                        pltpu.async_copy(src_of(t + 2), buf, sem)
                return 0
            lax.fori_loop(0, n_slots // 2, steps, 0)

            for b in range(2):
                pltpu.make_async_copy(
                    out_bufs[b], dst_of(n_slots - 2 + b), out_sems[b]).wait()

        plane_a = c * 24 + s
        zero_banks()
        hist_pass(plane_a, 0, n_chunks_full)
        build_lut()
        apply_pass(plane_a, 0, n_chunks_full)

        plane_b = c * 24 + 16 + s // 2
        half_base = (s % 2) * (H // 2)
        zero_banks()
        hist_pass(plane_b, 0, n_chunks_full)
        build_lut()
        apply_pass(plane_b, half_base, n_chunks_half)

    return eq_kernel


def kernel(x, magnitude):
    B, H, W, C = x.shape
    xp = jnp.transpose(x, (0, 3, 1, 2)).reshape(B * C, H, W)
    eq = _build(B * C, H, W)
    out = eq(xp.astype(jnp.float32))
    return out.reshape(B, C, H, W).transpose(0, 2, 3, 1).astype(x.dtype)
